# baseline (speedup 1.0000x reference)
import numpy as np
import jax
import jax.numpy as jnp
from jax.experimental import pallas as pl
from jax.experimental.pallas import tpu as pltpu

_K0 = np.uint32(0)
_K1 = np.uint32(1)
_K2 = np.uint32(int(_K0) ^ int(_K1) ^ 0x1BD11BDA)
_ROT = ((13, 15, 26, 6), (17, 29, 16, 24))

_B, _H, _W = 64, 1024, 512
_RB = 256
_GRID = (_B, _H // _RB)

_P1 = (2.81022636e-08, 3.43273939e-07, -3.5233877e-06, -4.39150654e-06,
       0.00021858087, -0.00125372503, -0.00417768164, 0.246640727, 1.50140941)
_P2 = (-0.000200214257, 0.000100950558, 0.00134934322, -0.00367342844,
       0.00573950773, -0.0076224613, 0.00943887047, 1.00167406, 2.83297682)
_U_LO = np.float32(np.nextafter(np.float32(-1.0), np.float32(0.0)))
_U_SCALE = np.float32(1.0) - _U_LO
_SQRT2 = np.float32(np.sqrt(np.float32(2.0)))


def _threefry_bits(idx):
    ks = (_K0, _K1, _K2)
    x0 = jnp.zeros_like(idx) + _K0
    x1 = idx + _K1
    for i in range(5):
        for r in _ROT[i % 2]:
            x0 = x0 + x1
            x1 = ((x1 << np.uint32(r)) | (x1 >> np.uint32(32 - r))) ^ x0
        x0 = x0 + ks[(i + 1) % 3]
        x1 = x1 + ks[(i + 2) % 3] + np.uint32(i + 1)
    return x0 ^ x1


def _std_normal(idx):
    bits = _threefry_bits(idx)
    f = jax.lax.bitcast_convert_type(
        (bits >> np.uint32(9)) | np.uint32(0x3F800000), jnp.float32)
    u = (f - np.float32(1.0)) * _U_SCALE + _U_LO
    u = jnp.maximum(_U_LO, u)
    w = -jnp.log1p(-u * u)
    w1 = w - np.float32(2.5)
    p1 = jnp.full_like(w, _P1[0])
    for c in _P1[1:]:
        p1 = p1 * w1 + np.float32(c)
    w2 = jnp.sqrt(w) - np.float32(3.0)
    p2 = jnp.full_like(w, _P2[0])
    for c in _P2[1:]:
        p2 = p2 * w2 + np.float32(c)
    p = jnp.where(w < np.float32(5.0), p1, p2)
    return _SQRT2 * p * u


def _noise_kernel(t_ref, a_ref, x0_ref, xt_ref, eps_ref):
    b = pl.program_id(0)
    rb = pl.program_id(1)
    abar = a_ref[t_ref[b]]
    s0 = jnp.sqrt(abar)
    s1 = jnp.sqrt(np.float32(1.0) - abar)
    base = (b * np.int32(_H * _W) + rb * np.int32(_RB * _W)).astype(jnp.uint32)
    row = jax.lax.broadcasted_iota(jnp.uint32, (1, _RB, _W), 1)
    col = jax.lax.broadcasted_iota(jnp.uint32, (1, _RB, _W), 2)
    idx = base + row * np.uint32(_W) + col
    eps = _std_normal(idx)
    eps_ref[...] = eps
    xt_ref[...] = s0 * x0_ref[...] + s1 * eps


def kernel(x0, t, alphas_cumprod):
    blk = pl.BlockSpec((1, _RB, _W), lambda b, r: (b, r, 0))
    out = jax.ShapeDtypeStruct((_B, _H, _W), jnp.float32)
    x_t, eps = pl.pallas_call(
        _noise_kernel,
        grid=_GRID,
        in_specs=[
            pl.BlockSpec(memory_space=pltpu.SMEM),
            pl.BlockSpec(memory_space=pltpu.SMEM),
            blk,
        ],
        out_specs=[blk, blk],
        out_shape=[out, out],
    )(t, alphas_cumprod, x0)
    return (x_t, eps)


# speedup vs baseline: 2.0973x; 2.0973x over previous
import numpy as np
import jax
import jax.numpy as jnp
from jax.experimental import pallas as pl
from jax.experimental.pallas import tpu as pltpu

_K0 = np.uint32(0)
_K1 = np.uint32(1)
_K2 = np.uint32(int(_K0) ^ int(_K1) ^ 0x1BD11BDA)
_ROT = ((13, 15, 26, 6), (17, 29, 16, 24))

_B, _H, _W = 64, 1024, 512
_RB = 256
_GRID = (_B, _H // _RB)

_U_LO = np.float32(np.nextafter(np.float32(-1.0), np.float32(0.0)))
_C = np.float32(2.5580564)
_PG = (-0.0063512907, 0.0011399789, 0.04059056, -0.06747421,
       0.03346732, 1.491577, 3.1025813)


def _threefry_bits(x1):
    ks = (_K0, _K1, _K2)
    x0 = x1
    x1 = ((x1 << np.uint32(13)) | (x1 >> np.uint32(19))) ^ x0
    for i in range(5):
        for r in _ROT[i % 2][1 if i == 0 else 0:]:
            x0 = x0 + x1
            x1 = ((x1 << np.uint32(r)) | (x1 >> np.uint32(32 - r))) ^ x0
        if int(ks[(i + 1) % 3]) != 0:
            x0 = x0 + ks[(i + 1) % 3]
        x1 = x1 + np.uint32((int(ks[(i + 2) % 3]) + i + 1) & 0xFFFFFFFF)
    return x0 ^ x1


def _std_normal(idx):
    bits = _threefry_bits(idx)
    f = jax.lax.bitcast_convert_type(
        (bits >> np.uint32(9)) | np.uint32(0x40000000), jnp.float32)
    u = jnp.maximum(_U_LO, f - np.float32(3.0))
    s = jnp.sqrt(np.float32(1.0) - jnp.log(np.float32(1.0) - u * u))
    y = s - _C
    p = jnp.full_like(y, _PG[0])
    for c in _PG[1:]:
        p = p * y + np.float32(c)
    return u * p


_RC = 128


def _noise_kernel(t_ref, a_ref, x0_ref, xt_ref, eps_ref):
    b = pl.program_id(0)
    rb = pl.program_id(1)
    abar = a_ref[t_ref[b]]
    s0 = jnp.sqrt(abar)
    s1 = jnp.sqrt(np.float32(1.0) - abar)
    base1 = (b * np.int32(_H * _W) + rb * np.int32(_RB * _W)
             + np.int32(int(_K1))).astype(jnp.uint32)
    row = jax.lax.broadcasted_iota(jnp.uint32, (1, _RC, _W), 1)
    col = jax.lax.broadcasted_iota(jnp.uint32, (1, _RC, _W), 2)
    local = row * np.uint32(_W) + col

    def body(c, _):
        sl = pl.ds(c * _RC, _RC)
        idx = (base1 + (c * np.int32(_RC * _W)).astype(jnp.uint32)) + local
        eps = _std_normal(idx)
        eps_ref[:, sl, :] = eps
        xt_ref[:, sl, :] = s0 * x0_ref[:, sl, :] + s1 * eps
        return _

    jax.lax.fori_loop(0, _RB // _RC, body, 0, unroll=False)


def kernel(x0, t, alphas_cumprod):
    blk = pl.BlockSpec((1, _RB, _W), lambda b, r: (b, r, 0))
    out = jax.ShapeDtypeStruct((_B, _H, _W), jnp.float32)
    x_t, eps = pl.pallas_call(
        _noise_kernel,
        grid=_GRID,
        in_specs=[
            pl.BlockSpec(memory_space=pltpu.SMEM),
            pl.BlockSpec(memory_space=pltpu.SMEM),
            blk,
        ],
        out_specs=[blk, blk],
        out_shape=[out, out],
    )(t, alphas_cumprod, x0)
    return (x_t, eps)
